# baseline (speedup 1.0000x reference)
import functools

import jax
import jax.numpy as jnp
from jax import lax
from jax.experimental import pallas as pl
from jax.experimental.pallas import tpu as pltpu
from jax.experimental.pallas import tpu_sc as plsc

NC = 2
NS = 16
NW = NC * NS
BATCH = 16384
EMB = 32
BPW = BATCH // NW
NCHUNK = 4
CHUNK = BPW // NCHUNK
LANES = 16


def _sc_gather(u_idx, p_idx, user_table, prodi_table, ub, pb):
    mesh = plsc.VectorSubcoreMesh(core_axis_name="c", subcore_axis_name="s")

    @functools.partial(
        pl.kernel,
        mesh=mesh,
        compiler_params=pltpu.CompilerParams(use_tc_tiling_on_sc=False),
        out_type=[
            jax.ShapeDtypeStruct((NW, LANES), jnp.float32),
            jax.ShapeDtypeStruct((BATCH,), jnp.float32),
        ],
        scratch_types=[
            pltpu.VMEM((NCHUNK, CHUNK), jnp.int32),
            pltpu.VMEM((NCHUNK, CHUNK), jnp.int32),
            pltpu.VMEM((BPW, EMB), jnp.float32),
            pltpu.VMEM((BPW, EMB), jnp.float32),
            pltpu.VMEM((BPW,), jnp.float32),
            pltpu.VMEM((BPW,), jnp.float32),
            pltpu.VMEM((BPW,), jnp.float32),
            pltpu.VMEM((LANES,), jnp.float32),
            pltpu.SemaphoreType.DMA,
            pltpu.SemaphoreType.DMA,
        ],
    )
    def k(u_idx_hbm, p_idx_hbm, ut_hbm, pt_hbm, ub_hbm, pb_hbm,
          partials_hbm, bsum_hbm,
          idx_u, idx_p, urows, prows, bu, bp, bsum, accv,
          sem_rows, sem_bias):
        wid = lax.axis_index("s") * NC + lax.axis_index("c")
        base = wid * BPW

        pltpu.sync_copy(u_idx_hbm.at[wid], idx_u)
        pltpu.sync_copy(p_idx_hbm.at[wid], idx_p)

        copies = []
        for j in range(NCHUNK):
            sl = pl.ds(j * CHUNK, CHUNK)
            copies.append(pltpu.async_copy(
                ut_hbm.at[idx_u.at[j]], urows.at[sl], sem_rows))
            copies.append(pltpu.async_copy(
                pt_hbm.at[idx_p.at[j]], prows.at[sl], sem_rows))
            copies.append(pltpu.async_copy(
                ub_hbm.at[idx_u.at[j]], bu.at[sl], sem_bias))
            copies.append(pltpu.async_copy(
                pb_hbm.at[idx_p.at[j]], bp.at[sl], sem_bias))
        for cp in copies:
            cp.wait()

        def bias_body(i, carry):
            s = pl.ds(pl.multiple_of(i * LANES, LANES), LANES)
            bsum[s] = bu[s] + bp[s]
            return carry
        lax.fori_loop(0, BPW // LANES, bias_body, 0)
        pltpu.sync_copy(bsum, bsum_hbm.at[pl.ds(base, BPW)])

        def dot_body(r, acc):
            a0 = urows[r, pl.ds(0, LANES)]
            a1 = urows[r, pl.ds(LANES, LANES)]
            b0 = prows[r, pl.ds(0, LANES)]
            b1 = prows[r, pl.ds(LANES, LANES)]
            return acc + a0 * b0 + a1 * b1
        acc = lax.fori_loop(0, BPW, dot_body,
                            jnp.zeros((LANES,), jnp.float32), unroll=4)
        accv[...] = acc
        pltpu.sync_copy(accv, partials_hbm.at[wid])

    return k(u_idx, p_idx, user_table, prodi_table, ub, pb)


def _finalize_body(partials_ref, bias_ref, out_ref):
    s = jnp.sum(partials_ref[...])
    x = bias_ref[...] + s
    out_ref[...] = 1.0 / (1.0 + jnp.exp(-x))


def kernel(inputs, user_table, user_bias_table, prodi_table, prodi_bias_table):
    u_idx = inputs[:, 0].reshape(NW, NCHUNK, CHUNK)
    p_idx = inputs[:, 1].reshape(NW, NCHUNK, CHUNK)
    ub = user_bias_table.reshape(-1)
    pb = prodi_bias_table.reshape(-1)

    partials, bsum = _sc_gather(u_idx, p_idx, user_table, prodi_table, ub, pb)

    out2d = pl.pallas_call(
        _finalize_body,
        out_shape=jax.ShapeDtypeStruct((128, 128), jnp.float32),
    )(partials, bsum.reshape(128, 128))
    return out2d.reshape(BATCH, 1)
